# baseline (speedup 1.0000x reference)
import jax
import jax.numpy as jnp
from jax.experimental import pallas as pl
from jax.experimental.pallas import tpu as pltpu


def _agg_body(c0_ref, c1_ref, c2_ref, e0_ref, e1_ref, e2_ref, o_ref):
    o_ref[...] = (
        e0_ref[...] * c0_ref[0]
        + e1_ref[...] * c1_ref[0]
        + e2_ref[...] * c2_ref[0]
    )


def kernel(embed_e0, embed_e1, embed_e2, coef_e0, coef_e1, coef_e2):
    N, D = embed_e0.shape
    B = 2000
    blk = pl.BlockSpec((B, D), lambda i: (i, 0))
    return pl.pallas_call(
        _agg_body,
        grid=(N // B,),
        in_specs=[
            pl.BlockSpec(memory_space=pltpu.SMEM),
            pl.BlockSpec(memory_space=pltpu.SMEM),
            pl.BlockSpec(memory_space=pltpu.SMEM),
            blk,
            blk,
            blk,
        ],
        out_specs=blk,
        out_shape=jax.ShapeDtypeStruct((N, D), embed_e0.dtype),
        compiler_params=pltpu.CompilerParams(
            dimension_semantics=("arbitrary",),
        ),
    )(coef_e0, coef_e1, coef_e2, embed_e0, embed_e1, embed_e2)


# speedup vs baseline: 1.0904x; 1.0904x over previous
import jax
import jax.numpy as jnp
from jax.experimental import pallas as pl
from jax.experimental.pallas import tpu as pltpu


def _agg_body(c0_ref, c1_ref, c2_ref, e0_ref, e1_ref, e2_ref, o_ref):
    o_ref[...] = (
        e0_ref[...] * c0_ref[0]
        + e1_ref[...] * c1_ref[0]
        + e2_ref[...] * c2_ref[0]
    )


def kernel(embed_e0, embed_e1, embed_e2, coef_e0, coef_e1, coef_e2):
    N, D = embed_e0.shape
    B = 5000
    blk = pl.BlockSpec((B, D), lambda i: (i, 0))
    return pl.pallas_call(
        _agg_body,
        grid=(N // B,),
        in_specs=[
            pl.BlockSpec(memory_space=pltpu.SMEM),
            pl.BlockSpec(memory_space=pltpu.SMEM),
            pl.BlockSpec(memory_space=pltpu.SMEM),
            blk,
            blk,
            blk,
        ],
        out_specs=blk,
        out_shape=jax.ShapeDtypeStruct((N, D), embed_e0.dtype),
        compiler_params=pltpu.CompilerParams(
            dimension_semantics=("arbitrary",),
        ),
    )(coef_e0, coef_e1, coef_e2, embed_e0, embed_e1, embed_e2)
